# baseline (speedup 1.0000x reference)
import jax
import jax.numpy as jnp
from jax.experimental import pallas as pl
from jax.experimental.pallas import tpu as pltpu

EPS_DICE = 0.0001
ROWS = 128
CHUNK = 8


def _dice_body(out_ref, tgt_ref, loss_ref, acc_ref):
    i = pl.program_id(1)
    nblk = pl.num_programs(1)
    c = out_ref.shape[1]
    w = out_ref.shape[3]
    nch = out_ref.shape[2] // CHUNK

    def chunk_step(j, carry):
        acc_sn, acc_t = carry
        x = out_ref[0, :, pl.ds(j * CHUNK, CHUNK), :]
        t = tgt_ref[0, pl.ds(j * CHUNK, CHUNK), :]
        sig = jax.nn.sigmoid(x)
        eq = jax.lax.broadcasted_iota(jnp.int32, x.shape, 0) == t[None]
        acc_sn = acc_sn + sig + eq.astype(jnp.float32)
        acc_t = acc_t + jnp.where(eq, sig, 0.0)
        return acc_sn, acc_t

    z = jnp.zeros((c, CHUNK, w), jnp.float32)
    acc_sn, acc_t = jax.lax.fori_loop(0, nch, chunk_step, (z, z))

    @pl.when(i == 0)
    def _init():
        acc_ref[...] = jnp.zeros_like(acc_ref)

    acc_ref[0] += jnp.sum(acc_sn, axis=1)
    acc_ref[1] += jnp.sum(acc_t, axis=1)

    @pl.when(i == nblk - 1)
    def _fin():
        sn = jnp.sum(acc_ref[0], axis=1)
        tt = jnp.sum(acc_ref[1], axis=1)
        per_class = 1.0 - 2.0 * tt / (sn + EPS_DICE)
        loss_ref[0, 0, :] = jnp.full((loss_ref.shape[-1],), jnp.sum(per_class) / c)


def kernel(output, target):
    b, c, h, w = output.shape
    tgt = target.astype(jnp.int32)
    nblk = h // ROWS
    padded = pl.pallas_call(
        _dice_body,
        grid=(b, nblk),
        in_specs=[
            pl.BlockSpec((1, c, ROWS, w), lambda bi, i: (bi, 0, i, 0)),
            pl.BlockSpec((1, ROWS, w), lambda bi, i: (bi, i, 0)),
        ],
        out_specs=pl.BlockSpec((1, 1, 128), lambda bi, i: (bi, 0, 0)),
        out_shape=jax.ShapeDtypeStruct((b, 1, 128), jnp.float32),
        scratch_shapes=[pltpu.VMEM((2, c, w), jnp.float32)],
    )(output, tgt)
    return padded[:, 0, 0]


# speedup vs baseline: 1.0725x; 1.0725x over previous
import jax
import jax.numpy as jnp
from jax.experimental import pallas as pl
from jax.experimental.pallas import tpu as pltpu

EPS_DICE = 0.0001
ROWS = 128
NC = 8


def _dice_body(*refs):
    xs = refs[:NC]
    tgt_ref = refs[NC]
    loss_ref = refs[NC + 1]
    acc_ref = refs[NC + 2]
    i = pl.program_id(1)
    nblk = pl.num_programs(1)
    t = tgt_ref[0]

    @pl.when(i == 0)
    def _init():
        acc_ref[...] = jnp.zeros_like(acc_ref)

    for c in range(NC):
        x = xs[c][0, 0]
        sig = jax.nn.sigmoid(x)
        eq = t == c
        acc_ref[0, c, :] += jnp.sum(sig + eq.astype(jnp.float32), axis=0)
        acc_ref[1, c, :] += jnp.sum(jnp.where(eq, sig, 0.0), axis=0)

    @pl.when(i == nblk - 1)
    def _fin():
        sn = jnp.sum(acc_ref[0], axis=1)
        tt = jnp.sum(acc_ref[1], axis=1)
        per_class = 1.0 - 2.0 * tt / (sn + EPS_DICE)
        loss_ref[0, 0, :] = jnp.full((loss_ref.shape[-1],), jnp.sum(per_class) / NC)


def kernel(output, target):
    b, nc, h, w = output.shape
    tgt = target.astype(jnp.int32)
    nblk = h // ROWS

    def chan_spec(c):
        return pl.BlockSpec((1, 1, ROWS, w), lambda bi, i, cc=c: (bi, cc, i, 0))

    padded = pl.pallas_call(
        _dice_body,
        grid=(b, nblk),
        in_specs=[chan_spec(c) for c in range(nc)]
        + [pl.BlockSpec((1, ROWS, w), lambda bi, i: (bi, i, 0))],
        out_specs=pl.BlockSpec((1, 1, 128), lambda bi, i: (bi, 0, 0)),
        out_shape=jax.ShapeDtypeStruct((b, 1, 128), jnp.float32),
        scratch_shapes=[pltpu.VMEM((2, nc, w), jnp.float32)],
    )(*([output] * nc + [tgt]))
    return padded[:, 0, 0]


# speedup vs baseline: 1.1295x; 1.0531x over previous
import jax
import jax.numpy as jnp
from jax.experimental import pallas as pl
from jax.experimental.pallas import tpu as pltpu

EPS_DICE = 0.0001
ROWS = 128
NC = 8


def _dice_body(*refs):
    xs = refs[:NC]
    tgt_ref = refs[NC]
    loss_ref = refs[NC + 1]
    acc_ref = refs[NC + 2]
    i = pl.program_id(1)
    nblk = pl.num_programs(1)
    out_shape_rows = tgt_ref.shape[1]
    t = tgt_ref[0]

    @pl.when(i == 0)
    def _init():
        acc_ref[...] = jnp.zeros_like(acc_ref)

    for c in range(NC):
        x = xs[c][0, 0]
        th = jnp.tanh(x * 0.5)
        eq = t == c
        acc_ref[0, c, :] += jnp.sum(th, axis=0)
        acc_ref[1, c, :] += jnp.sum(jnp.where(eq, th, 0.0), axis=0)
        acc_ref[2, c, :] += jnp.sum(jnp.where(eq, 1.0, 0.0), axis=0)

    @pl.when(i == nblk - 1)
    def _fin():
        npix = nblk * out_shape_rows * acc_ref.shape[2] * 1.0
        a = jnp.sum(acc_ref[0], axis=1)
        bb = jnp.sum(acc_ref[1], axis=1)
        n = jnp.sum(acc_ref[2], axis=1)
        s = 0.5 * (a + npix)
        tt = 0.5 * (bb + n)
        per_class = 1.0 - 2.0 * tt / (s + n + EPS_DICE)
        loss_ref[0, 0, :] = jnp.full((loss_ref.shape[-1],), jnp.sum(per_class) / NC)


def kernel(output, target):
    b, nc, h, w = output.shape
    tgt = target.astype(jnp.int32)
    nblk = h // ROWS

    def chan_spec(c):
        return pl.BlockSpec((1, 1, ROWS, w), lambda bi, i, cc=c: (bi, cc, i, 0))

    padded = pl.pallas_call(
        _dice_body,
        grid=(b, nblk),
        in_specs=[chan_spec(c) for c in range(nc)]
        + [pl.BlockSpec((1, ROWS, w), lambda bi, i: (bi, i, 0))],
        out_specs=pl.BlockSpec((1, 1, 128), lambda bi, i: (bi, 0, 0)),
        out_shape=jax.ShapeDtypeStruct((b, 1, 128), jnp.float32),
        scratch_shapes=[pltpu.VMEM((3, nc, w), jnp.float32)],
    )(*([output] * nc + [tgt]))
    return padded[:, 0, 0]


# speedup vs baseline: 1.6081x; 1.4237x over previous
import jax
import jax.numpy as jnp
from jax.experimental import pallas as pl
from jax.experimental.pallas import tpu as pltpu

EPS_DICE = 0.0001
NC = 8


def _dice_body(*refs):
    xs = refs[:NC]
    tgt_ref = refs[NC]
    loss_ref = refs[NC + 1]
    t = tgt_ref[0]
    h, w = t.shape
    npix = h * w * 1.0

    a_list = []
    bn_list = []
    n_list = []
    for c in range(NC):
        x = xs[c][0, 0]
        th = jnp.tanh(x * 0.5)
        eq = t == c
        a_list.append(jnp.sum(th, axis=0))
        bn_list.append(jnp.sum(jnp.where(eq, th, 0.0), axis=0))
        n_list.append(jnp.sum(jnp.where(eq, 1.0, 0.0), axis=0))

    a = jnp.stack([jnp.sum(v) for v in a_list])
    bb = jnp.stack([jnp.sum(v) for v in bn_list])
    n = jnp.stack([jnp.sum(v) for v in n_list])
    s = 0.5 * (a + npix)
    tt = 0.5 * (bb + n)
    per_class = 1.0 - 2.0 * tt / (s + n + EPS_DICE)
    loss_ref[0, 0, :] = jnp.full((loss_ref.shape[-1],), jnp.sum(per_class) / NC)


def kernel(output, target):
    b, nc, h, w = output.shape
    tgt = target.astype(jnp.int32)

    def chan_spec(c):
        return pl.BlockSpec((1, 1, h, w), lambda bi, cc=c: (bi, cc, 0, 0))

    padded = pl.pallas_call(
        _dice_body,
        grid=(b,),
        in_specs=[chan_spec(c) for c in range(nc)]
        + [pl.BlockSpec((1, h, w), lambda bi: (bi, 0, 0))],
        out_specs=pl.BlockSpec((1, 1, 128), lambda bi: (bi, 0, 0)),
        out_shape=jax.ShapeDtypeStruct((b, 1, 128), jnp.float32),
    )(*([output] * nc + [tgt]))
    return padded[:, 0, 0]


# speedup vs baseline: 1.6864x; 1.0487x over previous
import jax
import jax.numpy as jnp
from jax.experimental import pallas as pl
from jax.experimental.pallas import tpu as pltpu

EPS_DICE = 0.0001
NC = 8
BB = 2


def _dice_body(*refs):
    xs = refs[:NC]
    tgt_ref = refs[NC]
    loss_ref = refs[NC + 1]

    for b2 in range(BB):
        t = tgt_ref[b2]
        h, w = t.shape
        npix = h * w * 1.0

        a_list = []
        bn_list = []
        n_list = []
        for c in range(NC):
            x = xs[c][b2, 0]
            th = jnp.tanh(x * 0.5)
            eq = t == c
            a_list.append(jnp.sum(th, axis=0))
            bn_list.append(jnp.sum(jnp.where(eq, th, 0.0), axis=0))
            n_list.append(jnp.sum(jnp.where(eq, 1.0, 0.0), axis=0))

        a = jnp.stack([jnp.sum(v) for v in a_list])
        bb = jnp.stack([jnp.sum(v) for v in bn_list])
        n = jnp.stack([jnp.sum(v) for v in n_list])
        s = 0.5 * (a + npix)
        tt = 0.5 * (bb + n)
        per_class = 1.0 - 2.0 * tt / (s + n + EPS_DICE)
        loss_ref[b2, 0, :] = jnp.full((loss_ref.shape[-1],), jnp.sum(per_class) / NC)


def kernel(output, target):
    b, nc, h, w = output.shape
    tgt = target.astype(jnp.int32)

    def chan_spec(c):
        return pl.BlockSpec((BB, 1, h, w), lambda bi, cc=c: (bi, cc, 0, 0))

    padded = pl.pallas_call(
        _dice_body,
        grid=(b // BB,),
        in_specs=[chan_spec(c) for c in range(nc)]
        + [pl.BlockSpec((BB, h, w), lambda bi: (bi, 0, 0))],
        out_specs=pl.BlockSpec((BB, 1, 128), lambda bi: (bi, 0, 0)),
        out_shape=jax.ShapeDtypeStruct((b, 1, 128), jnp.float32),
    )(*([output] * nc + [tgt]))
    return padded[:, 0, 0]


# speedup vs baseline: 1.8432x; 1.0930x over previous
import jax
import jax.numpy as jnp
from jax.experimental import pallas as pl
from jax.experimental.pallas import tpu as pltpu

EPS_DICE = 0.0001
NC = 8
BB = 2


def _dice_body(*refs):
    xs = refs[:NC]
    tgt_ref = refs[NC]
    loss_ref = refs[NC + 1]

    for b2 in range(BB):
        t = tgt_ref[b2]
        h, w = t.shape
        npix = h * w * 1.0

        n_int = [jnp.zeros((w,), jnp.int32) for _ in range(NC)]
        qrows = 128
        for q in range(h // qrows):
            tq = t[q * qrows:(q + 1) * qrows, :]
            sh = jnp.left_shift(1, (tq & 3) << 3)
            is_lo = tq < 4
            lo = jnp.sum(jnp.where(is_lo, sh, 0), axis=0)
            hi = jnp.sum(jnp.where(is_lo, 0, sh), axis=0)
            for f in range(4):
                n_int[f] = n_int[f] + ((lo >> (8 * f)) & 255)
                n_int[4 + f] = n_int[4 + f] + ((hi >> (8 * f)) & 255)

        a_list = []
        bn_list = []
        for c in range(NC):
            x = xs[c][b2, 0]
            th = jnp.tanh(x * 0.5)
            eq = t == c
            a_list.append(jnp.sum(th, axis=0))
            bn_list.append(jnp.sum(jnp.where(eq, th, 0.0), axis=0))

        a = jnp.stack([jnp.sum(v) for v in a_list])
        bb = jnp.stack([jnp.sum(v) for v in bn_list])
        n = jnp.stack([jnp.sum(v).astype(jnp.float32) for v in n_int])
        s = 0.5 * (a + npix)
        tt = 0.5 * (bb + n)
        per_class = 1.0 - 2.0 * tt / (s + n + EPS_DICE)
        loss_ref[b2, 0, :] = jnp.full((loss_ref.shape[-1],), jnp.sum(per_class) / NC)


def kernel(output, target):
    b, nc, h, w = output.shape
    tgt = target.astype(jnp.int32)

    def chan_spec(c):
        return pl.BlockSpec((BB, 1, h, w), lambda bi, cc=c: (bi, cc, 0, 0))

    padded = pl.pallas_call(
        _dice_body,
        grid=(b // BB,),
        in_specs=[chan_spec(c) for c in range(nc)]
        + [pl.BlockSpec((BB, h, w), lambda bi: (bi, 0, 0))],
        out_specs=pl.BlockSpec((BB, 1, 128), lambda bi: (bi, 0, 0)),
        out_shape=jax.ShapeDtypeStruct((b, 1, 128), jnp.float32),
    )(*([output] * nc + [tgt]))
    return padded[:, 0, 0]
